# Initial kernel scaffold; baseline (speedup 1.0000x reference)
import jax
import jax.numpy as jnp
from jax.experimental import pallas as pl


def kernel(x, W_down, b_down, W_up, b_up, alpha, W_g1, b_g1, W_g2, b_g2):
    raise NotImplementedError("write your pallas kernel here")


# speedup vs baseline: 2.7040x; 2.7040x over previous
import functools

import jax
import jax.numpy as jnp
from jax.experimental import pallas as pl
from jax.experimental.pallas import tpu as pltpu

N = 4096
D = 512
BOT = 16
H = 256
C = 46
K = 10

RBLK = 256
NBLK = N // RBLK

_INTERPRET = False


def _split(a):
    hi = a.astype(jnp.bfloat16)
    lo = (a - hi.astype(jnp.float32)).astype(jnp.bfloat16)
    return hi, lo


def _dot3(ahi, alo, bhi, blo):
    acc = jax.lax.dot(ahi, bhi, preferred_element_type=jnp.float32)
    acc += jax.lax.dot(ahi, blo, preferred_element_type=jnp.float32)
    acc += jax.lax.dot(alo, bhi, preferred_element_type=jnp.float32)
    return acc


def _adapter_body(x_ref, wd_ref, bd_ref, wu_ref, bu_ref, alpha_ref, wg1_ref,
                  h_ref, sq_ref, nrm_ref, hhi_ref, hlo_ref, thi_ref, tlo_ref,
                  xl1_ref):
    x = x_ref[...]
    xhi, xlo = _split(x)
    wdhi, wdlo = _split(wd_ref[...])
    t = _dot3(xhi, xlo, wdhi, wdlo) + bd_ref[...]
    t = jnp.maximum(t, 0.0)
    thi, tlo = _split(t)
    wuhi, wulo = _split(wu_ref[...])
    u = _dot3(thi, tlo, wuhi, wulo) + bu_ref[...]
    h = alpha_ref[0, 0] * u + x
    h_ref[...] = h
    sq = jnp.sum(h * h, axis=1, keepdims=True)
    sq_ref[...] = sq
    nrm_ref[...] = jnp.sqrt(sq)
    hhi, hlo = _split(h)
    hhi_ref[...] = hhi
    hlo_ref[...] = hlo
    thi_ref[...] = hhi.T
    tlo_ref[...] = hlo.T
    wg1hi, wg1lo = _split(wg1_ref[...])
    xl1_ref[...] = _dot3(hhi, hlo, wg1hi, wg1lo)


def _knn_body(hhi_ref, hlo_ref, thi_ref, tlo_ref, sqc_ref, sqr_ref,
              idx_ref, val_ref):
    i = pl.program_id(0)
    g = _dot3(hhi_ref[...], hlo_ref[...], thi_ref[...], tlo_ref[...])
    nd = 2.0 * g - sqc_ref[...] - sqr_ref[...]
    col = jax.lax.broadcasted_iota(jnp.int32, (RBLK, N), 1)
    row = jax.lax.broadcasted_iota(jnp.int32, (RBLK, N), 0) + i * RBLK
    nd = jnp.where(col == row, -3e38, nd)
    idxs = []
    vals = []
    for _ in range(K):
        m = jnp.max(nd, axis=1, keepdims=True)
        sel = jnp.min(jnp.where(nd == m, col, N), axis=1, keepdims=True)
        nd = jnp.where(col == sel, -3e38, nd)
        idxs.append(sel[:, 0])
        vals.append(m[:, 0])
    idx_ref[...] = jnp.stack(idxs, axis=0)
    val_ref[...] = jnp.stack(vals, axis=0)


def _adapter_call(x, W_down, b_down, W_up, b_up, alpha, W_g1):
    full = lambda shape: pl.BlockSpec(shape, lambda i: (0,) * len(shape))
    return pl.pallas_call(
        _adapter_body,
        grid=(NBLK,),
        in_specs=[
            pl.BlockSpec((RBLK, D), lambda i: (i, 0)),
            full((D, BOT)),
            full((1, BOT)),
            full((BOT, D)),
            full((1, D)),
            full((1, 1)),
            full((D, H)),
        ],
        out_specs=[
            pl.BlockSpec((RBLK, D), lambda i: (i, 0)),
            pl.BlockSpec((RBLK, 1), lambda i: (i, 0)),
            pl.BlockSpec((RBLK, 1), lambda i: (i, 0)),
            pl.BlockSpec((RBLK, D), lambda i: (i, 0)),
            pl.BlockSpec((RBLK, D), lambda i: (i, 0)),
            pl.BlockSpec((D, RBLK), lambda i: (0, i)),
            pl.BlockSpec((D, RBLK), lambda i: (0, i)),
            pl.BlockSpec((RBLK, H), lambda i: (i, 0)),
        ],
        out_shape=[
            jax.ShapeDtypeStruct((N, D), jnp.float32),
            jax.ShapeDtypeStruct((N, 1), jnp.float32),
            jax.ShapeDtypeStruct((N, 1), jnp.float32),
            jax.ShapeDtypeStruct((N, D), jnp.bfloat16),
            jax.ShapeDtypeStruct((N, D), jnp.bfloat16),
            jax.ShapeDtypeStruct((D, N), jnp.bfloat16),
            jax.ShapeDtypeStruct((D, N), jnp.bfloat16),
            jax.ShapeDtypeStruct((N, H), jnp.float32),
        ],
        interpret=_INTERPRET,
    )(x, W_down, b_down.reshape(1, BOT), W_up, b_up.reshape(1, D),
      alpha.reshape(1, 1), W_g1)


def _knn_call(hhi, hlo, thi, tlo, sq):
    return pl.pallas_call(
        _knn_body,
        grid=(NBLK,),
        in_specs=[
            pl.BlockSpec((RBLK, D), lambda i: (i, 0)),
            pl.BlockSpec((RBLK, D), lambda i: (i, 0)),
            pl.BlockSpec((D, N), lambda i: (0, 0)),
            pl.BlockSpec((D, N), lambda i: (0, 0)),
            pl.BlockSpec((RBLK, 1), lambda i: (i, 0)),
            pl.BlockSpec((1, N), lambda i: (0, 0)),
        ],
        out_specs=[
            pl.BlockSpec((K, RBLK), lambda i: (0, i)),
            pl.BlockSpec((K, RBLK), lambda i: (0, i)),
        ],
        out_shape=[
            jax.ShapeDtypeStruct((K, N), jnp.int32),
            jax.ShapeDtypeStruct((K, N), jnp.float32),
        ],
        interpret=_INTERPRET,
    )(hhi, hlo, thi, tlo, sq, sq.reshape(1, N))


def kernel(x, W_down, b_down, W_up, b_up, alpha, W_g1, b_g1, W_g2, b_g2):
    x = x.astype(jnp.float32)
    (h, sq, nrm, hhi, hlo, thi, tlo, xl1) = _adapter_call(
        x, W_down, b_down, W_up, b_up, alpha, W_g1)
    idx_t, negd_t = _knn_call(hhi, hlo, thi, tlo, sq)

    sqv = sq[:, 0]
    nrmv = nrm[:, 0]
    sqj = sqv[idx_t]
    nrmj = nrmv[idx_t]
    gval = 0.5 * (sqv[None, :] + sqj + negd_t)
    ew = gval / jnp.maximum(nrmv[None, :] * nrmj, 1e-8)
    deg = 1.0 + jnp.sum(ew, axis=0)
    dis = jnp.where(deg > 0, deg ** -0.5, 0.0)
    w = dis[None, :] * ew * dis[idx_t]

    agg1 = jnp.einsum('kn,knh->nh', w, xl1[idx_t]) + (dis * dis)[:, None] * xl1
    y1 = jnp.maximum(agg1 + b_g1, 0.0)
    xl2 = y1 @ W_g2
    agg2 = jnp.einsum('kn,knc->nc', w, xl2[idx_t]) + (dis * dis)[:, None] * xl2
    g = jax.nn.softmax(agg2 + b_g2, axis=1)
    return (h, g)


# speedup vs baseline: 5.8671x; 2.1698x over previous
import functools

import jax
import jax.numpy as jnp
from jax import lax
from jax.experimental import pallas as pl
from jax.experimental.pallas import tpu as pltpu
from jax.experimental.pallas import tpu_sc as plsc

N = 4096
D = 512
BOT = 16
H = 256
C = 46
CP = 128
K = 10
KP = 12

RBLK = 256
NBLK = N // RBLK

NW = 32
NPW = N // NW
G = 16
NG = NPW // G

_INTERPRET = False


def _split(a):
    hi = a.astype(jnp.bfloat16)
    lo = (a - hi.astype(jnp.float32)).astype(jnp.bfloat16)
    return hi, lo


def _dot3(ahi, alo, bhi, blo):
    acc = jax.lax.dot(ahi, bhi, preferred_element_type=jnp.float32)
    acc += jax.lax.dot(ahi, blo, preferred_element_type=jnp.float32)
    acc += jax.lax.dot(alo, bhi, preferred_element_type=jnp.float32)
    return acc


def _dot3f(a, b):
    ahi, alo = _split(a)
    bhi, blo = _split(b)
    return _dot3(ahi, alo, bhi, blo)


def _adapter_body(x_ref, wd_ref, bd_ref, wu_ref, bu_ref, alpha_ref, wg1_ref,
                  h_ref, sq_ref, nrm_ref, hhi_ref, hlo_ref, thi_ref, tlo_ref,
                  xl1_ref):
    x = x_ref[...]
    t = jnp.maximum(_dot3f(x, wd_ref[...]) + bd_ref[...], 0.0)
    u = _dot3f(t, wu_ref[...]) + bu_ref[...]
    h = alpha_ref[0, 0] * u + x
    h_ref[...] = h
    sq = jnp.sum(h * h, axis=1, keepdims=True)
    sq_ref[...] = sq
    nrm_ref[...] = jnp.sqrt(sq)
    hhi, hlo = _split(h)
    hhi_ref[...] = hhi
    hlo_ref[...] = hlo
    thi_ref[...] = hhi.T
    tlo_ref[...] = hlo.T
    wg1hi, wg1lo = _split(wg1_ref[...])
    xl1_ref[...] = _dot3(hhi, hlo, wg1hi, wg1lo)


def _adapter_call(x, W_down, b_down, W_up, b_up, alpha, W_g1):
    full = lambda shape: pl.BlockSpec(shape, lambda i: (0,) * len(shape))
    return pl.pallas_call(
        _adapter_body,
        grid=(NBLK,),
        in_specs=[
            pl.BlockSpec((RBLK, D), lambda i: (i, 0)),
            full((D, BOT)),
            full((1, BOT)),
            full((BOT, D)),
            full((1, D)),
            full((1, 1)),
            full((D, H)),
        ],
        out_specs=[
            pl.BlockSpec((RBLK, D), lambda i: (i, 0)),
            pl.BlockSpec((RBLK, 1), lambda i: (i, 0)),
            pl.BlockSpec((RBLK, 1), lambda i: (i, 0)),
            pl.BlockSpec((RBLK, D), lambda i: (i, 0)),
            pl.BlockSpec((RBLK, D), lambda i: (i, 0)),
            pl.BlockSpec((D, RBLK), lambda i: (0, i)),
            pl.BlockSpec((D, RBLK), lambda i: (0, i)),
            pl.BlockSpec((RBLK, H), lambda i: (i, 0)),
        ],
        out_shape=[
            jax.ShapeDtypeStruct((N, D), jnp.float32),
            jax.ShapeDtypeStruct((N, 1), jnp.float32),
            jax.ShapeDtypeStruct((N, 1), jnp.float32),
            jax.ShapeDtypeStruct((N, D), jnp.bfloat16),
            jax.ShapeDtypeStruct((N, D), jnp.bfloat16),
            jax.ShapeDtypeStruct((D, N), jnp.bfloat16),
            jax.ShapeDtypeStruct((D, N), jnp.bfloat16),
            jax.ShapeDtypeStruct((N, H), jnp.float32),
        ],
        interpret=_INTERPRET,
    )(x, W_down, b_down.reshape(1, BOT), W_up, b_up.reshape(1, D),
      alpha.reshape(1, 1), W_g1)


def _knn_body(hhi_ref, hlo_ref, thi_ref, tlo_ref, sqc_ref, sqr_ref,
              idx_ref, val_ref):
    i = pl.program_id(0)
    g = _dot3(hhi_ref[...], hlo_ref[...], thi_ref[...], tlo_ref[...])
    nd = 2.0 * g - sqc_ref[...] - sqr_ref[...]
    col = jax.lax.broadcasted_iota(jnp.int32, (RBLK, N), 1)
    row = jax.lax.broadcasted_iota(jnp.int32, (RBLK, N), 0) + i * RBLK
    nd = jnp.where(col == row, -3e38, nd)
    idxs = []
    vals = []
    for _ in range(K):
        m = jnp.max(nd, axis=1, keepdims=True)
        sel = jnp.min(jnp.where(nd == m, col, N), axis=1, keepdims=True)
        nd = jnp.where(col == sel, -3e38, nd)
        idxs.append(sel[:, 0])
        vals.append(m[:, 0])
    idx_ref[...] = jnp.stack(idxs, axis=0)
    val_ref[...] = jnp.stack(vals, axis=0)


def _knn_call(hhi, hlo, thi, tlo, sq):
    return pl.pallas_call(
        _knn_body,
        grid=(NBLK,),
        in_specs=[
            pl.BlockSpec((RBLK, D), lambda i: (i, 0)),
            pl.BlockSpec((RBLK, D), lambda i: (i, 0)),
            pl.BlockSpec((D, N), lambda i: (0, 0)),
            pl.BlockSpec((D, N), lambda i: (0, 0)),
            pl.BlockSpec((RBLK, 1), lambda i: (i, 0)),
            pl.BlockSpec((1, N), lambda i: (0, 0)),
        ],
        out_specs=[
            pl.BlockSpec((K, RBLK), lambda i: (0, i)),
            pl.BlockSpec((K, RBLK), lambda i: (0, i)),
        ],
        out_shape=[
            jax.ShapeDtypeStruct((K, N), jnp.int32),
            jax.ShapeDtypeStruct((K, N), jnp.float32),
        ],
        interpret=_INTERPRET,
    )(hhi, hlo, thi, tlo, sq, sq.reshape(1, N))


_SC_MESH = plsc.VectorSubcoreMesh(core_axis_name="c", subcore_axis_name="s")
_SC_PARAMS = pltpu.CompilerParams(needs_layout_passes=False)


def _worker_base():
    wid = lax.axis_index("s") * 2 + lax.axis_index("c")
    return wid * NPW


def _rsqrt_bits(x):
    i = lax.bitcast_convert_type(x, jnp.int32)
    i = jnp.int32(0x5F3759DF) - lax.shift_right_arithmetic(i, 1)
    y = lax.bitcast_convert_type(i, jnp.float32)
    for _ in range(4):
        y = y * (1.5 - 0.5 * x * y * y)
    return y


def _iota16():
    return lax.iota(jnp.int32, 16)


@functools.partial(
    pl.kernel,
    mesh=_SC_MESH,
    compiler_params=_SC_PARAMS,
    out_type=[
        jax.ShapeDtypeStruct((K * N,), jnp.float32),
        jax.ShapeDtypeStruct((N,), jnp.float32),
    ],
    scratch_types=[
        pltpu.VMEM((N,), jnp.float32),
        pltpu.VMEM((N,), jnp.float32),
        pltpu.VMEM((K * NPW,), jnp.int32),
        pltpu.VMEM((K * NPW,), jnp.float32),
        pltpu.VMEM((K * NPW,), jnp.float32),
        pltpu.VMEM((NPW,), jnp.float32),
        pltpu.SemaphoreType.DMA,
    ],
)
def _edge_kernel(idx_hbm, negd_hbm, sq_hbm, nrm_hbm, ew_hbm, dis_hbm,
                 sq_v, nrm_v, idx_v, ngd_v, ew_v, dis_v, sem):
    base = _worker_base()
    pltpu.sync_copy(sq_hbm, sq_v)
    pltpu.sync_copy(nrm_hbm, nrm_v)
    for k in range(K):
        pltpu.sync_copy(idx_hbm.at[pl.ds(k * N + base, NPW)],
                        idx_v.at[pl.ds(k * NPW, NPW)])
        pltpu.sync_copy(negd_hbm.at[pl.ds(k * N + base, NPW)],
                        ngd_v.at[pl.ds(k * NPW, NPW)])
    for c in range(NPW // 16):
        sqi = sq_v[pl.ds(base + c * 16, 16)]
        nrmi = nrm_v[pl.ds(base + c * 16, 16)]
        deg = jnp.full((16,), 1.0, jnp.float32)
        for k in range(K):
            off = pl.ds(k * NPW + c * 16, 16)
            j = idx_v[off]
            sqj = plsc.load_gather(sq_v, [j])
            nrmj = plsc.load_gather(nrm_v, [j])
            gv = 0.5 * (sqi + sqj + ngd_v[off])
            ew = gv / jnp.maximum(nrmi * nrmj, 1e-8)
            ew_v[off] = ew
            deg = deg + ew
        y = _rsqrt_bits(deg)
        dis_v[pl.ds(c * 16, 16)] = jnp.where(deg > 0, y, 0.0)
    for k in range(K):
        pltpu.sync_copy(ew_v.at[pl.ds(k * NPW, NPW)],
                        ew_hbm.at[pl.ds(k * N + base, NPW)])
    pltpu.sync_copy(dis_v, dis_hbm.at[pl.ds(base, NPW)])


def _make_agg_kernel(Dv):
    @functools.partial(
        pl.kernel,
        mesh=_SC_MESH,
        compiler_params=_SC_PARAMS,
        out_type=jax.ShapeDtypeStruct((N, Dv), jnp.float32),
        scratch_types=[
            pltpu.VMEM((N,), jnp.float32),
            pltpu.VMEM((K * NPW,), jnp.int32),
            pltpu.VMEM((K * NPW,), jnp.float32),
            pltpu.VMEM((KP * NPW,), jnp.int32),
            pltpu.VMEM((KP * NPW,), jnp.float32),
            pltpu.VMEM((KP * G, Dv), jnp.float32),
            pltpu.VMEM((G, Dv), jnp.float32),
            pltpu.SemaphoreType.DMA,
            pltpu.SemaphoreType.DMA,
        ],
    )
    def _agg(xl_hbm, idx_hbm, ew_hbm, dis_hbm, out_hbm,
             dis_v, idx_v, ew_v, gidx_v, wts_v, rows_v, out_v, sem0, sem1):
        base = _worker_base()
        pltpu.sync_copy(dis_hbm, dis_v)
        for k in range(K):
            pltpu.sync_copy(idx_hbm.at[pl.ds(k * N + base, NPW)],
                            idx_v.at[pl.ds(k * NPW, NPW)])
            pltpu.sync_copy(ew_hbm.at[pl.ds(k * N + base, NPW)],
                            ew_v.at[pl.ds(k * NPW, NPW)])
        for c in range(NPW // 16):
            di = dis_v[pl.ds(base + c * 16, 16)]
            for k in range(KP):
                dst = pl.ds(c * (KP * 16) + k * 16, 16)
                if k < K:
                    off = pl.ds(k * NPW + c * 16, 16)
                    j = idx_v[off]
                    dj = plsc.load_gather(dis_v, [j])
                    gidx_v[dst] = j
                    wts_v[dst] = di * ew_v[off] * dj
                elif k == K:
                    gidx_v[dst] = base + c * 16 + _iota16()
                    wts_v[dst] = di * di
                else:
                    gidx_v[dst] = jnp.zeros((16,), jnp.int32)
                    wts_v[dst] = jnp.zeros((16,), jnp.float32)
        half = KP * G // 2
        for c in range(NG):
            g0 = c * (KP * 16)
            cp0 = pltpu.async_copy(
                xl_hbm.at[gidx_v.at[pl.ds(g0, half)]],
                rows_v.at[pl.ds(0, half)], sem0)
            cp1 = pltpu.async_copy(
                xl_hbm.at[gidx_v.at[pl.ds(g0 + half, half)]],
                rows_v.at[pl.ds(half, half)], sem1)
            cp0.wait()
            cp1.wait()

            def nbody(n, carry):
                wb = [plsc.load_gather(wts_v, [jnp.full((16,), g0 + k * 16 + n,
                                                        jnp.int32)])
                      for k in range(KP)]
                for f in range(Dv // 16):
                    fs = pl.ds(f * 16, 16)
                    acc = wb[0] * rows_v[n, fs]
                    for k in range(1, KP):
                        acc = acc + wb[k] * rows_v[k * 16 + n, fs]
                    out_v[n, fs] = acc
                return carry

            lax.fori_loop(0, G, nbody, 0)
            pltpu.sync_copy(out_v, out_hbm.at[pl.ds(base + c * G, G)])

    return _agg


_agg_h = _make_agg_kernel(H)
_agg_c = _make_agg_kernel(CP)


def _mid_body(agg_ref, bg1_ref, wg2_ref, out_ref):
    y = jnp.maximum(agg_ref[...] + bg1_ref[...], 0.0)
    out_ref[...] = _dot3f(y, wg2_ref[...])


def _mid_call(agg1, b_g1, W_g2p):
    return pl.pallas_call(
        _mid_body,
        grid=(8,),
        in_specs=[
            pl.BlockSpec((N // 8, H), lambda i: (i, 0)),
            pl.BlockSpec((1, H), lambda i: (0, 0)),
            pl.BlockSpec((H, CP), lambda i: (0, 0)),
        ],
        out_specs=pl.BlockSpec((N // 8, CP), lambda i: (i, 0)),
        out_shape=jax.ShapeDtypeStruct((N, CP), jnp.float32),
        interpret=_INTERPRET,
    )(agg1, b_g1.reshape(1, H), W_g2p)


def _smax_body(agg_ref, bg2_ref, out_ref):
    z = agg_ref[...] + bg2_ref[...]
    col = jax.lax.broadcasted_iota(jnp.int32, (N // 8, CP), 1)
    z = jnp.where(col >= C, -3e38, z)
    m = jnp.max(z, axis=1, keepdims=True)
    e = jnp.exp(z - m)
    s = jnp.sum(e, axis=1, keepdims=True)
    out_ref[...] = (e / s)[:, :C]


def _smax_call(agg2, b_g2p):
    return pl.pallas_call(
        _smax_body,
        grid=(8,),
        in_specs=[
            pl.BlockSpec((N // 8, CP), lambda i: (i, 0)),
            pl.BlockSpec((1, CP), lambda i: (0, 0)),
        ],
        out_specs=pl.BlockSpec((N // 8, C), lambda i: (i, 0)),
        out_shape=jax.ShapeDtypeStruct((N, C), jnp.float32),
        interpret=_INTERPRET,
    )(agg2, b_g2p)


def kernel(x, W_down, b_down, W_up, b_up, alpha, W_g1, b_g1, W_g2, b_g2):
    x = x.astype(jnp.float32)
    (h, sq, nrm, hhi, hlo, thi, tlo, xl1) = _adapter_call(
        x, W_down, b_down, W_up, b_up, alpha, W_g1)
    idx_t, negd_t = _knn_call(hhi, hlo, thi, tlo, sq)

    idx_f = idx_t.reshape(K * N)
    negd_f = negd_t.reshape(K * N)
    ew_f, dis = _edge_kernel(idx_f, negd_f, sq.reshape(N), nrm.reshape(N))

    agg1 = _agg_h(xl1, idx_f, ew_f, dis)
    W_g2p = jnp.pad(W_g2, ((0, 0), (0, CP - C)))
    xl2 = _mid_call(agg1, b_g1, W_g2p)
    agg2 = _agg_c(xl2, idx_f, ew_f, dis)
    b_g2p = jnp.pad(b_g2, (0, CP - C)).reshape(1, CP)
    g = _smax_call(agg2, b_g2p)
    return (h, g)


# speedup vs baseline: 9.3315x; 1.5905x over previous
import functools

import jax
import jax.numpy as jnp
from jax import lax
from jax.experimental import pallas as pl
from jax.experimental.pallas import tpu as pltpu
from jax.experimental.pallas import tpu_sc as plsc

N = 4096
D = 512
BOT = 16
H = 256
C = 46
CP = 128
K = 10
KP = 11

RBLK = 256
NBLK = N // RBLK

NW = 32
NPW = N // NW
G = 8
NG = NPW // G

_INTERPRET = False


def _split(a):
    hi = a.astype(jnp.bfloat16)
    lo = (a - hi.astype(jnp.float32)).astype(jnp.bfloat16)
    return hi, lo


def _dot3(ahi, alo, bhi, blo):
    acc = jax.lax.dot(ahi, bhi, preferred_element_type=jnp.float32)
    acc += jax.lax.dot(ahi, blo, preferred_element_type=jnp.float32)
    acc += jax.lax.dot(alo, bhi, preferred_element_type=jnp.float32)
    return acc


def _dot3f(a, b):
    ahi, alo = _split(a)
    bhi, blo = _split(b)
    return _dot3(ahi, alo, bhi, blo)


def _adapter_body(x_ref, wd_ref, bd_ref, wu_ref, bu_ref, alpha_ref, wg1_ref,
                  h_ref, sq_ref, nrm_ref, hhi_ref, hlo_ref, thi_ref, tlo_ref,
                  xl1_ref):
    x = x_ref[...]
    t = jnp.maximum(_dot3f(x, wd_ref[...]) + bd_ref[...], 0.0)
    u = _dot3f(t, wu_ref[...]) + bu_ref[...]
    h = alpha_ref[0, 0] * u + x
    h_ref[...] = h
    sq = jnp.sum(h * h, axis=1, keepdims=True)
    sq_ref[...] = sq
    nrm_ref[...] = jnp.sqrt(sq)
    hhi, hlo = _split(h)
    hhi_ref[...] = hhi
    hlo_ref[...] = hlo
    thi_ref[...] = hhi.T
    tlo_ref[...] = hlo.T
    wg1hi, wg1lo = _split(wg1_ref[...])
    xl1_ref[...] = _dot3(hhi, hlo, wg1hi, wg1lo)


def _adapter_call(x, W_down, b_down, W_up, b_up, alpha, W_g1):
    full = lambda shape: pl.BlockSpec(shape, lambda i: (0,) * len(shape))
    return pl.pallas_call(
        _adapter_body,
        grid=(NBLK,),
        in_specs=[
            pl.BlockSpec((RBLK, D), lambda i: (i, 0)),
            full((D, BOT)),
            full((1, BOT)),
            full((BOT, D)),
            full((1, D)),
            full((1, 1)),
            full((D, H)),
        ],
        out_specs=[
            pl.BlockSpec((RBLK, D), lambda i: (i, 0)),
            pl.BlockSpec((RBLK, 1), lambda i: (i, 0)),
            pl.BlockSpec((RBLK, 1), lambda i: (i, 0)),
            pl.BlockSpec((RBLK, D), lambda i: (i, 0)),
            pl.BlockSpec((RBLK, D), lambda i: (i, 0)),
            pl.BlockSpec((D, RBLK), lambda i: (0, i)),
            pl.BlockSpec((D, RBLK), lambda i: (0, i)),
            pl.BlockSpec((RBLK, H), lambda i: (i, 0)),
        ],
        out_shape=[
            jax.ShapeDtypeStruct((N, D), jnp.float32),
            jax.ShapeDtypeStruct((N, 1), jnp.float32),
            jax.ShapeDtypeStruct((N, 1), jnp.float32),
            jax.ShapeDtypeStruct((N, D), jnp.bfloat16),
            jax.ShapeDtypeStruct((N, D), jnp.bfloat16),
            jax.ShapeDtypeStruct((D, N), jnp.bfloat16),
            jax.ShapeDtypeStruct((D, N), jnp.bfloat16),
            jax.ShapeDtypeStruct((N, H), jnp.float32),
        ],
        interpret=_INTERPRET,
    )(x, W_down, b_down.reshape(1, BOT), W_up, b_up.reshape(1, D),
      alpha.reshape(1, 1), W_g1)


def _knn_body(hhi_ref, hlo_ref, thi_ref, tlo_ref, sqc_ref, sqr_ref,
              idx_ref, val_ref):
    i = pl.program_id(0)
    g = _dot3(hhi_ref[...], hlo_ref[...], thi_ref[...], tlo_ref[...])
    nd = 2.0 * g - sqc_ref[...] - sqr_ref[...]
    col = jax.lax.broadcasted_iota(jnp.int32, (RBLK, N), 1)
    row = jax.lax.broadcasted_iota(jnp.int32, (RBLK, N), 0) + i * RBLK
    nd = jnp.where(col == row, -3e38, nd)
    idxs = []
    vals = []
    for _ in range(K):
        m = jnp.max(nd, axis=1, keepdims=True)
        sel = jnp.min(jnp.where(nd == m, col, N), axis=1, keepdims=True)
        nd = jnp.where(col == sel, -3e38, nd)
        idxs.append(sel[:, 0])
        vals.append(m[:, 0])
    idx_ref[...] = jnp.stack(idxs, axis=0)
    val_ref[...] = jnp.stack(vals, axis=0)


def _knn_call(hhi, hlo, thi, tlo, sq):
    return pl.pallas_call(
        _knn_body,
        grid=(NBLK,),
        in_specs=[
            pl.BlockSpec((RBLK, D), lambda i: (i, 0)),
            pl.BlockSpec((RBLK, D), lambda i: (i, 0)),
            pl.BlockSpec((D, N), lambda i: (0, 0)),
            pl.BlockSpec((D, N), lambda i: (0, 0)),
            pl.BlockSpec((RBLK, 1), lambda i: (i, 0)),
            pl.BlockSpec((1, N), lambda i: (0, 0)),
        ],
        out_specs=[
            pl.BlockSpec((K, RBLK), lambda i: (0, i)),
            pl.BlockSpec((K, RBLK), lambda i: (0, i)),
        ],
        out_shape=[
            jax.ShapeDtypeStruct((K, N), jnp.int32),
            jax.ShapeDtypeStruct((K, N), jnp.float32),
        ],
        interpret=_INTERPRET,
    )(hhi, hlo, thi, tlo, sq, sq.reshape(1, N))


_SC_MESH = plsc.VectorSubcoreMesh(core_axis_name="c", subcore_axis_name="s")
_SC_PARAMS = pltpu.CompilerParams(needs_layout_passes=False)


def _worker_base():
    wid = lax.axis_index("s") * 2 + lax.axis_index("c")
    return wid * NPW


def _rsqrt_bits(x):
    i = lax.bitcast_convert_type(x, jnp.int32)
    i = jnp.int32(0x5F3759DF) - lax.shift_right_arithmetic(i, 1)
    y = lax.bitcast_convert_type(i, jnp.float32)
    for _ in range(4):
        y = y * (1.5 - 0.5 * x * y * y)
    return y


def _iota16():
    return lax.iota(jnp.int32, 16)


@functools.partial(
    pl.kernel,
    mesh=_SC_MESH,
    compiler_params=_SC_PARAMS,
    out_type=[
        jax.ShapeDtypeStruct((K * N,), jnp.float32),
        jax.ShapeDtypeStruct((N,), jnp.float32),
    ],
    scratch_types=[
        pltpu.VMEM((N,), jnp.float32),
        pltpu.VMEM((N,), jnp.float32),
        pltpu.VMEM((K * NPW,), jnp.int32),
        pltpu.VMEM((K * NPW,), jnp.float32),
        pltpu.VMEM((K * NPW,), jnp.float32),
        pltpu.VMEM((NPW,), jnp.float32),
        pltpu.SemaphoreType.DMA,
    ],
)
def _edge_kernel(idx_hbm, negd_hbm, sq_hbm, nrm_hbm, ew_hbm, dis_hbm,
                 sq_v, nrm_v, idx_v, ngd_v, ew_v, dis_v, sem):
    base = _worker_base()
    pltpu.sync_copy(sq_hbm, sq_v)
    pltpu.sync_copy(nrm_hbm, nrm_v)
    for k in range(K):
        pltpu.sync_copy(idx_hbm.at[pl.ds(k * N + base, NPW)],
                        idx_v.at[pl.ds(k * NPW, NPW)])
        pltpu.sync_copy(negd_hbm.at[pl.ds(k * N + base, NPW)],
                        ngd_v.at[pl.ds(k * NPW, NPW)])
    for c in range(NPW // 16):
        sqi = sq_v[pl.ds(base + c * 16, 16)]
        nrmi = nrm_v[pl.ds(base + c * 16, 16)]
        deg = jnp.full((16,), 1.0, jnp.float32)
        for k in range(K):
            off = pl.ds(k * NPW + c * 16, 16)
            j = idx_v[off]
            sqj = plsc.load_gather(sq_v, [j])
            nrmj = plsc.load_gather(nrm_v, [j])
            gv = 0.5 * (sqi + sqj + ngd_v[off])
            ew = gv / jnp.maximum(nrmi * nrmj, 1e-8)
            ew_v[off] = ew
            deg = deg + ew
        y = _rsqrt_bits(deg)
        dis_v[pl.ds(c * 16, 16)] = jnp.where(deg > 0, y, 0.0)
    for k in range(K):
        pltpu.sync_copy(ew_v.at[pl.ds(k * NPW, NPW)],
                        ew_hbm.at[pl.ds(k * N + base, NPW)])
    pltpu.sync_copy(dis_v, dis_hbm.at[pl.ds(base, NPW)])


def _make_agg_kernel(Dv):
    @functools.partial(
        pl.kernel,
        mesh=_SC_MESH,
        compiler_params=_SC_PARAMS,
        out_type=jax.ShapeDtypeStruct((N, Dv), jnp.float32),
        scratch_types=[
            pltpu.VMEM((N,), jnp.float32),
            pltpu.VMEM((K * NPW,), jnp.int32),
            pltpu.VMEM((K * NPW,), jnp.float32),
            pltpu.VMEM((KP * NPW,), jnp.int32),
            pltpu.VMEM((KP * NPW,), jnp.float32),
            pltpu.VMEM((KP * G, Dv), jnp.float32),
            pltpu.VMEM((KP * G, Dv), jnp.float32),
            pltpu.VMEM((G, Dv), jnp.float32),
            pltpu.SemaphoreType.DMA,
            pltpu.SemaphoreType.DMA,
        ],
    )
    def _agg(xl_hbm, idx_hbm, ew_hbm, dis_hbm, out_hbm,
             dis_v, idx_v, ew_v, gidx_v, wts_v, rows_a, rows_b, out_v,
             sem_a, sem_b):
        base = _worker_base()
        pltpu.sync_copy(dis_hbm, dis_v)
        for k in range(K):
            pltpu.sync_copy(idx_hbm.at[pl.ds(k * N + base, NPW)],
                            idx_v.at[pl.ds(k * NPW, NPW)])
            pltpu.sync_copy(ew_hbm.at[pl.ds(k * N + base, NPW)],
                            ew_v.at[pl.ds(k * NPW, NPW)])
        ii = _iota16()
        grp = lax.shift_right_logical(ii, 3) * (KP * G)
        lane = jnp.bitwise_and(ii, 7)
        for c in range(NPW // 16):
            di = dis_v[pl.ds(base + c * 16, 16)]
            for k in range(KP):
                dst = 2 * c * (KP * G) + grp + k * G + lane
                if k < K:
                    off = pl.ds(k * NPW + c * 16, 16)
                    j = idx_v[off]
                    dj = plsc.load_gather(dis_v, [j])
                    plsc.store_scatter(gidx_v, [dst], j)
                    plsc.store_scatter(wts_v, [dst], di * ew_v[off] * dj)
                else:
                    plsc.store_scatter(gidx_v, [dst], base + c * 16 + ii)
                    plsc.store_scatter(wts_v, [dst], di * di)

        nrows = KP * G

        def issue(g, buf, sem):
            return pltpu.async_copy(
                xl_hbm.at[gidx_v.at[pl.ds(g * nrows, nrows)]], buf, sem)

        def wait(g, buf, sem):
            pltpu.make_async_copy(
                xl_hbm.at[gidx_v.at[pl.ds(g * nrows, nrows)]], buf, sem).wait()

        def compute(g, buf):
            for ni in range(G):
                wb = [plsc.load_gather(
                    wts_v, [jnp.full((16,), g * nrows + k * G + ni, jnp.int32)])
                    for k in range(KP)]
                for f in range(Dv // 16):
                    fs = pl.ds(f * 16, 16)
                    acc = wb[0] * buf[ni, fs]
                    for k in range(1, KP):
                        acc = acc + wb[k] * buf[k * G + ni, fs]
                    out_v[ni, fs] = acc
            pltpu.sync_copy(out_v, out_hbm.at[pl.ds(base + g * G, G)])

        issue(0, rows_a, sem_a)

        def body(t, carry):
            g0 = 2 * t
            issue(g0 + 1, rows_b, sem_b)
            wait(g0, rows_a, sem_a)
            compute(g0, rows_a)

            @pl.when(t < NG // 2 - 1)
            def _():
                issue(g0 + 2, rows_a, sem_a)

            wait(g0 + 1, rows_b, sem_b)
            compute(g0 + 1, rows_b)
            return carry

        lax.fori_loop(0, NG // 2, body, 0)

    return _agg


_agg_h = _make_agg_kernel(H)
_agg_c = _make_agg_kernel(CP)


def _mid_body(agg_ref, bg1_ref, wg2_ref, out_ref):
    y = jnp.maximum(agg_ref[...] + bg1_ref[...], 0.0)
    out_ref[...] = _dot3f(y, wg2_ref[...])


def _mid_call(agg1, b_g1, W_g2p):
    return pl.pallas_call(
        _mid_body,
        grid=(8,),
        in_specs=[
            pl.BlockSpec((N // 8, H), lambda i: (i, 0)),
            pl.BlockSpec((1, H), lambda i: (0, 0)),
            pl.BlockSpec((H, CP), lambda i: (0, 0)),
        ],
        out_specs=pl.BlockSpec((N // 8, CP), lambda i: (i, 0)),
        out_shape=jax.ShapeDtypeStruct((N, CP), jnp.float32),
        interpret=_INTERPRET,
    )(agg1, b_g1.reshape(1, H), W_g2p)


def _smax_body(agg_ref, bg2_ref, out_ref):
    z = agg_ref[...] + bg2_ref[...]
    col = jax.lax.broadcasted_iota(jnp.int32, (N // 8, CP), 1)
    z = jnp.where(col >= C, -3e38, z)
    m = jnp.max(z, axis=1, keepdims=True)
    e = jnp.exp(z - m)
    s = jnp.sum(e, axis=1, keepdims=True)
    out_ref[...] = (e / s)[:, :C]


def _smax_call(agg2, b_g2p):
    return pl.pallas_call(
        _smax_body,
        grid=(8,),
        in_specs=[
            pl.BlockSpec((N // 8, CP), lambda i: (i, 0)),
            pl.BlockSpec((1, CP), lambda i: (0, 0)),
        ],
        out_specs=pl.BlockSpec((N // 8, C), lambda i: (i, 0)),
        out_shape=jax.ShapeDtypeStruct((N, C), jnp.float32),
        interpret=_INTERPRET,
    )(agg2, b_g2p)


def kernel(x, W_down, b_down, W_up, b_up, alpha, W_g1, b_g1, W_g2, b_g2):
    x = x.astype(jnp.float32)
    (h, sq, nrm, hhi, hlo, thi, tlo, xl1) = _adapter_call(
        x, W_down, b_down, W_up, b_up, alpha, W_g1)
    idx_t, negd_t = _knn_call(hhi, hlo, thi, tlo, sq)

    idx_f = idx_t.reshape(K * N)
    negd_f = negd_t.reshape(K * N)
    ew_f, dis = _edge_kernel(idx_f, negd_f, sq.reshape(N), nrm.reshape(N))

    agg1 = _agg_h(xl1, idx_f, ew_f, dis)
    W_g2p = jnp.pad(W_g2, ((0, 0), (0, CP - C)))
    xl2 = _mid_call(agg1, b_g1, W_g2p)
    agg2 = _agg_c(xl2, idx_f, ew_f, dis)
    b_g2p = jnp.pad(b_g2, (0, CP - C)).reshape(1, CP)
    g = _smax_call(agg2, b_g2p)
    return (h, g)
